# baseline (speedup 1.0000x reference)
import functools

import jax
import jax.numpy as jnp
from jax import lax
from jax.experimental import pallas as pl
from jax.experimental.pallas import tpu as pltpu
from jax.experimental.pallas import tpu_sc as plsc

NUM_EMB = 8192
DIM = 256
ROW_TILE = 1024


LANE = 128


def _argmin_body(x_ref, emb_ref, idx_ref, esq_ref):
    i = pl.program_id(0)

    @pl.when(i == 0)
    def _():
        e = emb_ref[...]
        esq_ref[...] = jnp.sum(e * e, axis=0, keepdims=True)

    x = x_ref[...]
    sim2 = jnp.dot(x * (-2.0), emb_ref[...],
                   preferred_element_type=jnp.float32)
    xsq = jnp.sum(x * x, axis=1, keepdims=True)
    esq = esq_ref[...]

    n_chunks = NUM_EMB // LANE
    best = jnp.full((ROW_TILE, LANE), jnp.inf, jnp.float32)
    bestc = jnp.zeros((ROW_TILE, LANE), jnp.float32)
    for c in range(n_chunks):
        sl = slice(c * LANE, (c + 1) * LANE)
        d_c = (xsq + esq[:, sl]) + sim2[:, sl]
        take = d_c < best
        best = jnp.minimum(best, d_c)
        bestc = jnp.where(take, jnp.float32(c), bestc)

    m = jnp.min(best, axis=1, keepdims=True)
    lane = lax.broadcasted_iota(jnp.int32, (ROW_TILE, LANE), 1).astype(jnp.float32)
    colf = bestc * jnp.float32(LANE) + lane
    idxf = jnp.min(jnp.where(best == m, colf, jnp.float32(3e38)),
                   axis=1, keepdims=True)
    idx_ref[...] = idxf.astype(jnp.int32).reshape(1, ROW_TILE, 1)


def _codebook_argmin(flat, embeddings):
    n_rows = flat.shape[0]
    grid = (n_rows // ROW_TILE,)
    idx3 = pl.pallas_call(
        _argmin_body,
        grid=grid,
        in_specs=[
            pl.BlockSpec((ROW_TILE, DIM), lambda i: (i, 0)),
            pl.BlockSpec((DIM, NUM_EMB), lambda i: (0, 0)),
        ],
        out_specs=pl.BlockSpec((1, ROW_TILE, 1), lambda i: (i, 0, 0)),
        out_shape=jax.ShapeDtypeStruct((grid[0], ROW_TILE, 1), jnp.int32),
        scratch_shapes=[
            pltpu.VMEM((1, NUM_EMB), jnp.float32),
        ],
        compiler_params=pltpu.CompilerParams(
            dimension_semantics=("arbitrary",),
        ),
    )(flat, embeddings)
    return idx3.reshape(n_rows)


def _make_sc_gather(n_rows):
    info = plsc.get_sparse_core_info()
    nw = info.num_cores * info.num_subcores
    b_per_w = n_rows // nw
    mesh = plsc.VectorSubcoreMesh(core_axis_name="c", subcore_axis_name="s")

    @functools.partial(
        pl.kernel, mesh=mesh,
        out_type=jax.ShapeDtypeStruct((n_rows, DIM), jnp.float32),
        scratch_types=[
            pltpu.VMEM((b_per_w,), jnp.int32),
            pltpu.VMEM((b_per_w, DIM), jnp.float32),
            pltpu.SemaphoreType.DMA,
        ],
    )
    def gather(table_hbm, idx_hbm, out_hbm, idx_v, rows_v, sem):
        wid = lax.axis_index("s") * info.num_cores + lax.axis_index("c")
        base = wid * b_per_w
        pltpu.sync_copy(idx_hbm.at[pl.ds(base, b_per_w)], idx_v)
        pltpu.async_copy(table_hbm.at[idx_v], rows_v, sem).wait()
        pltpu.sync_copy(rows_v, out_hbm.at[pl.ds(base, b_per_w)])

    return gather


def kernel(x, embeddings):
    B, T, D = x.shape
    flat = x.reshape(B * T, D)
    idx = _codebook_argmin(flat, embeddings)
    quantized = jnp.take(embeddings.T, idx, axis=0)
    return quantized.reshape(B, T, D)


# speedup vs baseline: 1.1198x; 1.1198x over previous
import functools

import jax
import jax.numpy as jnp
from jax import lax
from jax.experimental import pallas as pl
from jax.experimental.pallas import tpu as pltpu
from jax.experimental.pallas import tpu_sc as plsc

NUM_EMB = 8192
DIM = 256
ROW_TILE = 1024


LANE = 128


def _argmin_body(x_ref, emb_ref, idx_ref, esq_ref):
    i = pl.program_id(0)

    @pl.when(i == 0)
    def _():
        e = emb_ref[...]
        esq_ref[...] = jnp.sum(e * e, axis=0, keepdims=True)

    x = x_ref[...]
    sim2 = jnp.dot(x * (-2.0), emb_ref[...],
                   preferred_element_type=jnp.float32)
    xsq = jnp.sum(x * x, axis=1, keepdims=True)
    esq = esq_ref[...]

    n_chunks = NUM_EMB // LANE
    best = jnp.full((ROW_TILE, LANE), jnp.inf, jnp.float32)
    bestc = jnp.zeros((ROW_TILE, LANE), jnp.float32)
    for c in range(n_chunks):
        sl = slice(c * LANE, (c + 1) * LANE)
        d_c = (xsq + esq[:, sl]) + sim2[:, sl]
        take = d_c < best
        best = jnp.minimum(best, d_c)
        bestc = jnp.where(take, jnp.float32(c), bestc)

    m = jnp.min(best, axis=1, keepdims=True)
    lane = lax.broadcasted_iota(jnp.int32, (ROW_TILE, LANE), 1).astype(jnp.float32)
    colf = bestc * jnp.float32(LANE) + lane
    idxf = jnp.min(jnp.where(best == m, colf, jnp.float32(3e38)),
                   axis=1, keepdims=True)
    idx_ref[...] = idxf.astype(jnp.int32).T.reshape(1, 1, ROW_TILE)


def _codebook_argmin(flat, embeddings):
    n_rows = flat.shape[0]
    grid = (n_rows // ROW_TILE,)
    idx3 = pl.pallas_call(
        _argmin_body,
        grid=grid,
        in_specs=[
            pl.BlockSpec((ROW_TILE, DIM), lambda i: (i, 0)),
            pl.BlockSpec((DIM, NUM_EMB), lambda i: (0, 0)),
        ],
        out_specs=pl.BlockSpec((1, 1, ROW_TILE), lambda i: (i, 0, 0)),
        out_shape=jax.ShapeDtypeStruct((grid[0], 1, ROW_TILE), jnp.int32),
        scratch_shapes=[
            pltpu.VMEM((1, NUM_EMB), jnp.float32),
        ],
        compiler_params=pltpu.CompilerParams(
            dimension_semantics=("arbitrary",),
        ),
    )(flat, embeddings)
    return idx3.reshape(n_rows)


def _make_sc_gather(n_rows):
    info = plsc.get_sparse_core_info()
    nw = info.num_cores * info.num_subcores
    b_per_w = n_rows // nw
    mesh = plsc.VectorSubcoreMesh(core_axis_name="c", subcore_axis_name="s")

    @functools.partial(
        pl.kernel, mesh=mesh,
        out_type=jax.ShapeDtypeStruct((n_rows, DIM), jnp.float32),
        scratch_types=[
            pltpu.VMEM((b_per_w,), jnp.int32),
            pltpu.VMEM((b_per_w, DIM), jnp.float32),
            pltpu.SemaphoreType.DMA,
        ],
    )
    def gather(table_hbm, idx_hbm, out_hbm, idx_v, rows_v, sem):
        wid = lax.axis_index("s") * info.num_cores + lax.axis_index("c")
        base = wid * b_per_w
        pltpu.sync_copy(idx_hbm.at[pl.ds(base, b_per_w)], idx_v)
        pltpu.async_copy(table_hbm.at[idx_v], rows_v, sem).wait()
        pltpu.sync_copy(rows_v, out_hbm.at[pl.ds(base, b_per_w)])

    return gather


def kernel(x, embeddings):
    B, T, D = x.shape
    flat = x.reshape(B * T, D)
    idx = _codebook_argmin(flat, embeddings)
    table = embeddings.T
    quantized = _make_sc_gather(B * T)(table, idx)
    return quantized.reshape(B, T, D)
